# Initial kernel scaffold; baseline (speedup 1.0000x reference)
import jax
import jax.numpy as jnp
from jax.experimental import pallas as pl


def kernel(x, edge_index, W1, b1, W2, b2, W3, b3):
    raise NotImplementedError("write your pallas kernel here")


# speedup vs baseline: 3.9264x; 3.9264x over previous
import functools

import jax
import jax.numpy as jnp
from jax import lax
from jax.experimental import pallas as pl
from jax.experimental.pallas import tpu as pltpu
from jax.experimental.pallas import tpu_sc as plsc

NC = 2
NS = 16
LANES = 16

N_PAD = 10240
E_PAD = 163840
EB = 128
NB = E_PAD // (NS * EB)
DH = 512
DC = 128
NCHUNK = DH // DC
RPT = N_PAD // NS
BN = 256


def _sc_mesh():
    return plsc.VectorSubcoreMesh(
        core_axis_name="c", subcore_axis_name="s", num_cores=NC, num_subcores=NS
    )


def _fill_rows(buf, nrows, width, vec):
    for r in range(nrows):
        for i in range(width // LANES):
            buf[r, pl.ds(i * LANES, LANES)] = vec


def _zero_accum_slice(zbuf, zrows, accum, base, nrows):
    off = 0
    while off < nrows:
        step = min(zrows, nrows - off)
        if step == zrows:
            pltpu.sync_copy(zbuf, accum.at[pl.ds(base + off, step)])
        else:
            pltpu.sync_copy(zbuf.at[pl.ds(0, step)], accum.at[pl.ds(base + off, step)])
        off += step


def _deg_body(col_hbm, out_hbm, colbuf, onesbuf, zbuf, accum):
    c = lax.axis_index("c")
    s = lax.axis_index("s")
    _fill_rows(zbuf, 64, LANES, jnp.zeros((LANES,), jnp.float32))
    _fill_rows(onesbuf, EB, LANES, jnp.ones((LANES,), jnp.float32))
    _zero_accum_slice(zbuf, 64, accum, s * RPT, RPT)
    pltpu.sync_copy(col_hbm.at[s].at[pl.ds(c * (NB // NC), NB // NC)], colbuf)
    plsc.subcore_barrier()

    @pl.loop(0, NB // NC)
    def _(j):
        pltpu.sync_copy(onesbuf, accum.at[colbuf.at[j]], add=True)

    plsc.subcore_barrier()
    pltpu.sync_copy(
        accum.at[pl.ds(s * RPT, RPT)], out_hbm.at[c].at[pl.ds(s * RPT, RPT)]
    )


def _deg_counts(col3):
    k = pl.kernel(
        _deg_body,
        out_type=jax.ShapeDtypeStruct((NC, N_PAD, LANES), jnp.float32),
        mesh=_sc_mesh(),
        scratch_types=[
            pltpu.VMEM((NB // NC, EB), jnp.int32),
            pltpu.VMEM((EB, LANES), jnp.float32),
            pltpu.VMEM((64, LANES), jnp.float32),
            pltpu.VMEM_SHARED((N_PAD, LANES), jnp.float32),
        ],
    )
    return k(col3)


def _scatter_body(g_hbm, row_hbm, col_hbm, out_hbm,
                  rowbuf, colbuf, gbuf, zbuf, accum, sem):
    c = lax.axis_index("c")
    s = lax.axis_index("s")
    _fill_rows(zbuf, 64, DC, jnp.zeros((LANES,), jnp.float32))
    pltpu.sync_copy(row_hbm.at[s], rowbuf)
    pltpu.sync_copy(col_hbm.at[s], colbuf)
    for kk in range(NCHUNK // NC):
        chunk = c * (NCHUNK // NC) + kk
        _zero_accum_slice(zbuf, 64, accum, s * RPT, RPT)
        plsc.subcore_barrier()

        @pl.loop(0, NB)
        def _(j):
            pltpu.async_copy(g_hbm.at[chunk].at[rowbuf.at[j]], gbuf, sem).wait()
            pltpu.sync_copy(gbuf, accum.at[colbuf.at[j]], add=True)

        plsc.subcore_barrier()
        pltpu.sync_copy(
            accum.at[pl.ds(s * RPT, RPT)],
            out_hbm.at[chunk].at[pl.ds(s * RPT, RPT)],
        )
        plsc.subcore_barrier()


def _edge_scatter(g, row3, col3):
    k = pl.kernel(
        _scatter_body,
        out_type=jax.ShapeDtypeStruct((NCHUNK, N_PAD, DC), jnp.float32),
        mesh=_sc_mesh(),
        scratch_types=[
            pltpu.VMEM((NB, EB), jnp.int32),
            pltpu.VMEM((NB, EB), jnp.int32),
            pltpu.VMEM((EB, DC), jnp.float32),
            pltpu.VMEM((64, DC), jnp.float32),
            pltpu.VMEM_SHARED((N_PAD, DC), jnp.float32),
            pltpu.SemaphoreType.DMA,
        ],
    )
    return k(g, row3, col3)


def _dinv(degp_ref):
    p = degp_ref[...]
    cnt = p[0, :, 0] + p[1, :, 0]
    return lax.rsqrt(cnt + 1.0)


def _m1_body(degp_ref, x_ref, w_ref, b_ref, g_ref, u_ref):
    dinv = _dinv(degp_ref)
    h = (
        jnp.dot(x_ref[...], w_ref[...], preferred_element_type=jnp.float32)
        + b_ref[...]
    )
    g_ref[0] = dinv[:, None] * h
    u_ref[...] = (dinv * dinv)[:, None] * h


def _m1(degp, x_p, w_t, b_r):
    d_in = x_p.shape[1]
    return pl.pallas_call(
        _m1_body,
        grid=(N_PAD // BN, DH // DC),
        in_specs=[
            pl.BlockSpec((NC, BN, LANES), lambda i, j: (0, i, 0)),
            pl.BlockSpec((BN, d_in), lambda i, j: (i, 0)),
            pl.BlockSpec((d_in, DC), lambda i, j: (0, j)),
            pl.BlockSpec((1, DC), lambda i, j: (0, j)),
        ],
        out_specs=[
            pl.BlockSpec((1, BN, DC), lambda i, j: (j, i, 0)),
            pl.BlockSpec((BN, DC), lambda i, j: (i, j)),
        ],
        out_shape=[
            jax.ShapeDtypeStruct((NCHUNK, N_PAD, DC), jnp.float32),
            jax.ShapeDtypeStruct((N_PAD, DH), jnp.float32),
        ],
    )(degp, x_p, w_t, b_r)


def _m2_body(degp_ref, s_ref, u_ref, w_ref, b_ref, g_ref, u2_ref, acc):
    k = pl.program_id(2)
    dinv = _dinv(degp_ref)
    d2 = dinv * dinv
    z = jnp.maximum(dinv[:, None] * s_ref[0] + d2[:, None] * u_ref[...], 0.0)

    @pl.when(k == 0)
    def _():
        acc[...] = jnp.zeros_like(acc)

    acc[...] += jnp.dot(z, w_ref[...], preferred_element_type=jnp.float32)

    @pl.when(k == NCHUNK - 1)
    def _():
        h = acc[...] + b_ref[...]
        g_ref[0] = dinv[:, None] * h
        u2_ref[...] = d2[:, None] * h


def _m2(degp, s1, u1, w_t, b_r):
    return pl.pallas_call(
        _m2_body,
        grid=(N_PAD // BN, DH // DC, NCHUNK),
        in_specs=[
            pl.BlockSpec((NC, BN, LANES), lambda i, j, k: (0, i, 0)),
            pl.BlockSpec((1, BN, DC), lambda i, j, k: (k, i, 0)),
            pl.BlockSpec((BN, DC), lambda i, j, k: (i, k)),
            pl.BlockSpec((DC, DC), lambda i, j, k: (k, j)),
            pl.BlockSpec((1, DC), lambda i, j, k: (0, j)),
        ],
        out_specs=[
            pl.BlockSpec((1, BN, DC), lambda i, j, k: (j, i, 0)),
            pl.BlockSpec((BN, DC), lambda i, j, k: (i, j)),
        ],
        out_shape=[
            jax.ShapeDtypeStruct((NCHUNK, N_PAD, DC), jnp.float32),
            jax.ShapeDtypeStruct((N_PAD, DH), jnp.float32),
        ],
        scratch_shapes=[pltpu.VMEM((BN, DC), jnp.float32)],
    )(degp, s1, u1, w_t, b_r)


def _m3_body(degp_ref, s_ref, u_ref, w_ref, b_ref, o_ref, acc):
    k = pl.program_id(1)
    dinv = _dinv(degp_ref)
    d2 = dinv * dinv
    z = jnp.maximum(dinv[:, None] * s_ref[0] + d2[:, None] * u_ref[...], 0.0)

    @pl.when(k == 0)
    def _():
        acc[...] = jnp.zeros_like(acc)

    acc[...] += jnp.dot(z, w_ref[...], preferred_element_type=jnp.float32)

    @pl.when(k == NCHUNK - 1)
    def _():
        o_ref[...] = acc[...] + b_ref[...]


def _m3(degp, s2, u2, w_t, b_r):
    d_out = w_t.shape[1]
    return pl.pallas_call(
        _m3_body,
        grid=(N_PAD // BN, NCHUNK),
        in_specs=[
            pl.BlockSpec((NC, BN, LANES), lambda i, k: (0, i, 0)),
            pl.BlockSpec((1, BN, DC), lambda i, k: (k, i, 0)),
            pl.BlockSpec((BN, DC), lambda i, k: (i, k)),
            pl.BlockSpec((DC, d_out), lambda i, k: (k, 0)),
            pl.BlockSpec((1, d_out), lambda i, k: (0, 0)),
        ],
        out_specs=pl.BlockSpec((BN, d_out), lambda i, k: (i, 0)),
        out_shape=jax.ShapeDtypeStruct((N_PAD, d_out), jnp.float32),
        scratch_shapes=[pltpu.VMEM((BN, d_out), jnp.float32)],
    )(degp, s2, u2, w_t, b_r)


def kernel(x, edge_index, W1, b1, W2, b2, W3, b3):
    n, _ = x.shape
    e = edge_index.shape[1]
    x_p = jnp.pad(x, ((0, N_PAD - n), (0, 0)))
    row3 = jnp.pad(edge_index[0], (0, E_PAD - e)).reshape(NS, NB, EB)
    col3 = jnp.pad(edge_index[1], (0, E_PAD - e), constant_values=n).reshape(
        NS, NB, EB
    )

    degp = _deg_counts(col3)
    g1, u1 = _m1(degp, x_p, W1.T, b1.reshape(1, -1))
    s1 = _edge_scatter(g1, row3, col3)
    g2, u2 = _m2(degp, s1, u1, W2.T, b2.reshape(1, -1))
    s2 = _edge_scatter(g2, row3, col3)
    y = _m3(degp, s2, u2, W3.T, b3.reshape(1, -1))
    return y[:n]
